# Initial kernel scaffold; baseline (speedup 1.0000x reference)
import jax
import jax.numpy as jnp
from jax.experimental import pallas as pl


def kernel(z, mu, epoch):
    raise NotImplementedError("write your pallas kernel here")


# speedup vs baseline: 79.0181x; 79.0181x over previous
import functools

import jax
import jax.numpy as jnp
from jax.experimental import pallas as pl
from jax.experimental.pallas import tpu as pltpu
import numpy as np

EPS = 1e-6
TOPK = 5
SIGMA_HI = 1e4
CALIB_ITERS = 30 * TOPK

_VMEM = 64 * 1024 * 1024


def _cparams(n_par):
    return pltpu.CompilerParams(
        dimension_semantics=("parallel",) * n_par,
        vmem_limit_bytes=_VMEM,
    )


def _dist_block(zb, mut):
    dh = zb.shape[1]
    cross = jnp.dot(zb, mut, preferred_element_type=jnp.float32)
    z2 = jnp.sum(zb * zb, axis=1, keepdims=True)
    zs = jnp.sum(zb, axis=1, keepdims=True)
    m2 = jnp.sum(mut * mut, axis=0, keepdims=True)
    ms = jnp.sum(mut, axis=0, keepdims=True)
    d2 = z2 + m2 - 2.0 * cross + 2.0 * EPS * (zs - ms) + dh * (EPS * EPS)
    return jnp.sqrt(jnp.maximum(d2, 0.0))


def _extract_min(work, idx, axis, dim):
    m = jnp.min(work, axis=axis, keepdims=True)
    ismin = work == m
    idxf = jnp.min(jnp.where(ismin, idx, dim), axis=axis, keepdims=True)
    nwork = jnp.where(idx == idxf, jnp.float32(jnp.inf), work)
    return m, nwork


def _topk_kernel(z_ref, mut_ref, relz_ref, colp_ref, *, kk):
    d = _dist_block(z_ref[...], mut_ref[...])
    bn, kc = d.shape
    lane_idx = jax.lax.broadcasted_iota(jnp.int32, d.shape, 1)
    work = d
    rels = []
    for j in range(kk):
        m, work = _extract_min(work, lane_idx, 1, kc)
        rels.append(m)
    relz_ref[...] = jnp.concatenate(rels, axis=1)
    row_idx = jax.lax.broadcasted_iota(jnp.int32, d.shape, 0)
    work = d
    cols = []
    for j in range(kk):
        m, work = _extract_min(work, row_idx, 0, bn)
        cols.append(m)
    pad = jnp.full((8 - kk, kc), jnp.inf, jnp.float32)
    colp_ref[...] = jnp.concatenate(cols + [pad], axis=0).reshape(1, 8, kc)


def _calib_loop(a, target, iters):
    shp = a.shape[1:]
    lo = jnp.zeros(shp, jnp.float32)
    hi = jnp.full(shp, SIGMA_HI, jnp.float32)
    sig = jnp.ones(shp, jnp.float32)

    def body(_, carry):
        lo, hi, sig = carry
        r = 1.0 / sig
        cur = jnp.ones(shp, jnp.float32)
        for j in range(1, a.shape[0]):
            cur = cur + jnp.exp(-(a[j] * r))
        gt = cur > target
        lo = jnp.where(gt, lo, sig)
        hi = jnp.where(gt, sig, hi)
        sig = (lo + hi) * 0.5
        return lo, hi, sig

    _, _, sig = jax.lax.fori_loop(0, iters, body, (lo, hi, sig))
    return sig


def _calib_z_kernel(relt_ref, rsig_ref, *, target, iters):
    rel = relt_ref[...]
    a = jax.nn.relu(rel - rel[0:1])
    sig = _calib_loop(a, target, iters)
    rsig_ref[...] = 1.0 / sig


def _calib_u_kernel(colp_ref, uvec_ref, *, kk, target1, iters):
    cp = colp_ref[...]
    nb, _, kc = cp.shape
    work = cp.reshape(nb * 8, kc)
    row_idx = jax.lax.broadcasted_iota(jnp.int32, work.shape, 0)
    rels = []
    for j in range(kk):
        m, work = _extract_min(work, row_idx, 0, nb * 8)
        rels.append(m)
    rel = jnp.concatenate(rels, axis=0)
    a = jax.nn.relu(rel - rel[0:1]).reshape(kk, 1, kc)
    sig = _calib_loop(a, target1, iters)
    uvec_ref[...] = jnp.concatenate([rels[0], 1.0 / sig], axis=0)


def _ws_kernel(z_ref, mut_ref, rhoz_ref, rsigz_ref, uvec_ref,
               w1_ref, s_ref, colsum_ref):
    d = _dist_block(z_ref[...], mut_ref[...])
    kc = d.shape[1]
    w1 = jnp.exp(-(jax.nn.relu(d - rhoz_ref[...]) * rsigz_ref[...]))
    w2 = jnp.exp(-(jax.nn.relu(d - uvec_ref[0:1, :]) * uvec_ref[1:2, :]))
    s = w1 + w2 - w1 * w2
    s = s * (1.0 / jnp.sum(s, axis=1, keepdims=True))
    w1_ref[...] = w1
    s_ref[...] = s
    colsum_ref[...] = jnp.sum(s, axis=0, keepdims=True).reshape(1, 1, kc)


def _dmat_kernel(s_ref, colp_ref, out_ref):
    s = s_ref[...]
    cs = jnp.sum(colp_ref[...], axis=0)
    dn = (s * s) * (1.0 / cs)
    out_ref[...] = dn * (1.0 / jnp.sum(dn, axis=1, keepdims=True))


@jax.jit
def kernel(z, mu, epoch):
    n, dh = z.shape
    kc = mu.shape[0]
    kk = min(TOPK, kc)
    f32 = jnp.float32

    bn = min(2048, n)
    nb = n // bn
    mut = mu.T

    relz, colp = pl.pallas_call(
        functools.partial(_topk_kernel, kk=kk),
        grid=(nb,),
        in_specs=[
            pl.BlockSpec((bn, dh), lambda i: (i, 0)),
            pl.BlockSpec((dh, kc), lambda i: (0, 0)),
        ],
        out_specs=[
            pl.BlockSpec((bn, kk), lambda i: (i, 0)),
            pl.BlockSpec((1, 8, kc), lambda i: (i, 0, 0)),
        ],
        out_shape=[
            jax.ShapeDtypeStruct((n, kk), f32),
            jax.ShapeDtypeStruct((nb, 8, kc), f32),
        ],
        compiler_params=_cparams(1),
        name="lgc_topk",
    )(z, mut)

    g = n // 128
    bg = min(64, g)
    relt = relz.T.reshape(kk, g, 128)
    target = np.float32(np.log2(kk) - 1.0)
    target1 = np.float32(np.log2(kk))
    rsigz = pl.pallas_call(
        functools.partial(_calib_z_kernel, target=target, iters=CALIB_ITERS),
        grid=(g // bg,),
        in_specs=[pl.BlockSpec((kk, bg, 128), lambda i: (0, i, 0))],
        out_specs=pl.BlockSpec((bg, 128), lambda i: (i, 0)),
        out_shape=jax.ShapeDtypeStruct((g, 128), f32),
        compiler_params=_cparams(1),
        name="lgc_calib_z",
    )(relt).reshape(n, 1)
    rhoz = relz[:, :1]

    uvec = pl.pallas_call(
        functools.partial(_calib_u_kernel, kk=kk, target1=target1,
                          iters=CALIB_ITERS),
        out_shape=jax.ShapeDtypeStruct((2, kc), f32),
        name="lgc_calib_u",
    )(colp)

    w1, s, colps = pl.pallas_call(
        _ws_kernel,
        grid=(nb,),
        in_specs=[
            pl.BlockSpec((bn, dh), lambda i: (i, 0)),
            pl.BlockSpec((dh, kc), lambda i: (0, 0)),
            pl.BlockSpec((bn, 1), lambda i: (i, 0)),
            pl.BlockSpec((bn, 1), lambda i: (i, 0)),
            pl.BlockSpec((2, kc), lambda i: (0, 0)),
        ],
        out_specs=[
            pl.BlockSpec((bn, kc), lambda i: (i, 0)),
            pl.BlockSpec((bn, kc), lambda i: (i, 0)),
            pl.BlockSpec((1, 1, kc), lambda i: (i, 0, 0)),
        ],
        out_shape=[
            jax.ShapeDtypeStruct((n, kc), f32),
            jax.ShapeDtypeStruct((n, kc), f32),
            jax.ShapeDtypeStruct((nb, 1, kc), f32),
        ],
        compiler_params=_cparams(1),
        name="lgc_ws",
    )(z, mut, rhoz, rsigz, uvec)

    dmat = pl.pallas_call(
        _dmat_kernel,
        grid=(nb,),
        in_specs=[
            pl.BlockSpec((bn, kc), lambda i: (i, 0)),
            pl.BlockSpec((nb, 1, kc), lambda i: (0, 0, 0)),
        ],
        out_specs=pl.BlockSpec((bn, kc), lambda i: (i, 0)),
        out_shape=jax.ShapeDtypeStruct((n, kc), f32),
        compiler_params=_cparams(1),
        name="lgc_dmat",
    )(s, colps)

    return (w1, s, w1, dmat)


# speedup vs baseline: 101.4456x; 1.2838x over previous
import functools

import jax
import jax.numpy as jnp
from jax.experimental import pallas as pl
from jax.experimental.pallas import tpu as pltpu
import numpy as np

EPS = 1e-6
TOPK = 5
SIGMA_HI = 1e4
CALIB_ITERS = 64

_VMEM = 64 * 1024 * 1024


def _cparams(n_par):
    return pltpu.CompilerParams(
        dimension_semantics=("parallel",) * n_par,
        vmem_limit_bytes=_VMEM,
    )


def _dist_block(zb, mut):
    dh = zb.shape[1]
    cross = jnp.dot(zb, mut, preferred_element_type=jnp.float32)
    z2 = jnp.sum(zb * zb, axis=1, keepdims=True)
    zs = jnp.sum(zb, axis=1, keepdims=True)
    m2 = jnp.sum(mut * mut, axis=0, keepdims=True)
    ms = jnp.sum(mut, axis=0, keepdims=True)
    d2 = z2 + m2 - 2.0 * cross + 2.0 * EPS * (zs - ms) + dh * (EPS * EPS)
    return jnp.sqrt(jnp.maximum(d2, 0.0))


def _extract_min(work, idx, axis, dim):
    m = jnp.min(work, axis=axis, keepdims=True)
    ismin = work == m
    idxf = jnp.min(jnp.where(ismin, idx, jnp.float32(dim)), axis=axis,
                   keepdims=True)
    nwork = jnp.where(idx == idxf, jnp.float32(jnp.inf), work)
    return m, nwork


def _topk_kernel(z_ref, mut_ref, relz_ref, colp_ref, *, kk):
    d = _dist_block(z_ref[...], mut_ref[...])
    bn, kc = d.shape
    lane_idx = jax.lax.broadcasted_iota(jnp.int32, d.shape, 1).astype(jnp.float32)
    work = d
    rels = []
    for j in range(kk):
        m, work = _extract_min(work, lane_idx, 1, kc)
        rels.append(m)
    relz_ref[...] = jnp.concatenate(rels, axis=1)
    d3 = d.reshape(bn // 8, 8, kc)
    runs = [jnp.full((8, kc), jnp.inf, jnp.float32) for _ in range(kk)]
    for i in range(bn // 8):
        c = d3[i]
        for j in range(kk):
            lo = jnp.minimum(runs[j], c)
            if j < kk - 1:
                c = jnp.maximum(runs[j], c)
            runs[j] = lo
    cand = jnp.concatenate(runs, axis=0)
    row_idx = jax.lax.broadcasted_iota(jnp.int32, cand.shape, 0).astype(jnp.float32)
    work = cand
    cols = []
    for j in range(kk):
        m, work = _extract_min(work, row_idx, 0, 8 * kk)
        cols.append(m)
    pad = jnp.full((8 - kk, kc), jnp.inf, jnp.float32)
    colp_ref[...] = jnp.concatenate(cols + [pad], axis=0).reshape(1, 8, kc)


def _calib_loop(a, target, iters):
    shp = a.shape[1:]
    lo = jnp.zeros(shp, jnp.float32)
    hi = jnp.full(shp, SIGMA_HI, jnp.float32)
    sig = jnp.ones(shp, jnp.float32)

    def body(_, carry):
        lo, hi, sig = carry
        r = 1.0 / sig
        cur = jnp.ones(shp, jnp.float32)
        for j in range(1, a.shape[0]):
            cur = cur + jnp.exp(-(a[j] * r))
        gt = cur > target
        lo = jnp.where(gt, lo, sig)
        hi = jnp.where(gt, sig, hi)
        sig = (lo + hi) * 0.5
        return lo, hi, sig

    _, _, sig = jax.lax.fori_loop(0, iters, body, (lo, hi, sig))
    return sig


def _calib_z_kernel(relt_ref, rsig_ref, *, target, iters):
    rel = relt_ref[...]
    a = jax.nn.relu(rel - rel[0:1])
    sig = _calib_loop(a, target, iters)
    rsig_ref[...] = 1.0 / sig


def _calib_u_kernel(colp_ref, uvec_ref, *, kk, target1, iters):
    cp = colp_ref[...]
    nb, _, kc = cp.shape
    work = cp.reshape(nb * 8, kc)
    row_idx = jax.lax.broadcasted_iota(jnp.int32, work.shape, 0).astype(jnp.float32)
    rels = []
    for j in range(kk):
        m, work = _extract_min(work, row_idx, 0, nb * 8)
        rels.append(m)
    rel = jnp.concatenate(rels, axis=0)
    a = jax.nn.relu(rel - rel[0:1]).reshape(kk, 1, kc)
    sig = _calib_loop(a, target1, iters)
    uvec_ref[...] = jnp.concatenate([rels[0], 1.0 / sig], axis=0)


def _ws_kernel(z_ref, mut_ref, rhoz_ref, rsigz_ref, uvec_ref,
               w1_ref, s_ref, colsum_ref):
    d = _dist_block(z_ref[...], mut_ref[...])
    kc = d.shape[1]
    w1 = jnp.exp(-(jax.nn.relu(d - rhoz_ref[...]) * rsigz_ref[...]))
    w2 = jnp.exp(-(jax.nn.relu(d - uvec_ref[0:1, :]) * uvec_ref[1:2, :]))
    s = w1 + w2 - w1 * w2
    s = s * (1.0 / jnp.sum(s, axis=1, keepdims=True))
    w1_ref[...] = w1
    s_ref[...] = s
    colsum_ref[...] = jnp.sum(s, axis=0, keepdims=True).reshape(1, 1, kc)


def _dmat_kernel(s_ref, colp_ref, out_ref):
    s = s_ref[...]
    cs = jnp.sum(colp_ref[...], axis=0)
    dn = (s * s) * (1.0 / cs)
    out_ref[...] = dn * (1.0 / jnp.sum(dn, axis=1, keepdims=True))


@jax.jit
def kernel(z, mu, epoch):
    n, dh = z.shape
    kc = mu.shape[0]
    kk = min(TOPK, kc)
    f32 = jnp.float32

    bn = min(2048, n)
    nb = n // bn
    mut = mu.T

    relz, colp = pl.pallas_call(
        functools.partial(_topk_kernel, kk=kk),
        grid=(nb,),
        in_specs=[
            pl.BlockSpec((bn, dh), lambda i: (i, 0)),
            pl.BlockSpec((dh, kc), lambda i: (0, 0)),
        ],
        out_specs=[
            pl.BlockSpec((bn, kk), lambda i: (i, 0)),
            pl.BlockSpec((1, 8, kc), lambda i: (i, 0, 0)),
        ],
        out_shape=[
            jax.ShapeDtypeStruct((n, kk), f32),
            jax.ShapeDtypeStruct((nb, 8, kc), f32),
        ],
        compiler_params=_cparams(1),
        name="lgc_topk",
    )(z, mut)

    g = n // 128
    bg = min(64, g)
    relt = relz.T.reshape(kk, g, 128)
    target = np.float32(np.log2(kk) - 1.0)
    target1 = np.float32(np.log2(kk))
    rsigz = pl.pallas_call(
        functools.partial(_calib_z_kernel, target=target, iters=CALIB_ITERS),
        grid=(g // bg,),
        in_specs=[pl.BlockSpec((kk, bg, 128), lambda i: (0, i, 0))],
        out_specs=pl.BlockSpec((bg, 128), lambda i: (i, 0)),
        out_shape=jax.ShapeDtypeStruct((g, 128), f32),
        compiler_params=_cparams(1),
        name="lgc_calib_z",
    )(relt).reshape(n, 1)
    rhoz = relz[:, :1]

    uvec = pl.pallas_call(
        functools.partial(_calib_u_kernel, kk=kk, target1=target1,
                          iters=CALIB_ITERS),
        out_shape=jax.ShapeDtypeStruct((2, kc), f32),
        name="lgc_calib_u",
    )(colp)

    w1, s, colps = pl.pallas_call(
        _ws_kernel,
        grid=(nb,),
        in_specs=[
            pl.BlockSpec((bn, dh), lambda i: (i, 0)),
            pl.BlockSpec((dh, kc), lambda i: (0, 0)),
            pl.BlockSpec((bn, 1), lambda i: (i, 0)),
            pl.BlockSpec((bn, 1), lambda i: (i, 0)),
            pl.BlockSpec((2, kc), lambda i: (0, 0)),
        ],
        out_specs=[
            pl.BlockSpec((bn, kc), lambda i: (i, 0)),
            pl.BlockSpec((bn, kc), lambda i: (i, 0)),
            pl.BlockSpec((1, 1, kc), lambda i: (i, 0, 0)),
        ],
        out_shape=[
            jax.ShapeDtypeStruct((n, kc), f32),
            jax.ShapeDtypeStruct((n, kc), f32),
            jax.ShapeDtypeStruct((nb, 1, kc), f32),
        ],
        compiler_params=_cparams(1),
        name="lgc_ws",
    )(z, mut, rhoz, rsigz, uvec)

    dmat = pl.pallas_call(
        _dmat_kernel,
        grid=(nb,),
        in_specs=[
            pl.BlockSpec((bn, kc), lambda i: (i, 0)),
            pl.BlockSpec((nb, 1, kc), lambda i: (0, 0, 0)),
        ],
        out_specs=pl.BlockSpec((bn, kc), lambda i: (i, 0)),
        out_shape=jax.ShapeDtypeStruct((n, kc), f32),
        compiler_params=_cparams(1),
        name="lgc_dmat",
    )(s, colps)

    return (w1, s, w1, dmat)
